# baseline (speedup 1.0000x reference)
import functools

import jax
import jax.numpy as jnp
from jax import lax
from jax.experimental import pallas as pl
from jax.experimental.pallas import tpu as pltpu
from jax.experimental.pallas import tpu_sc as plsc

_R = 32
_D = 1024


def kernel(input, pe_table):
    B, S, D = input.shape
    x1 = input.reshape(B * S * D)
    pe1 = pe_table.reshape(pe_table.shape[0] * D)
    NW = 32
    rows_per_w = S // NW
    chunk = _R * D
    mesh = plsc.VectorSubcoreMesh(core_axis_name="c", subcore_axis_name="s")

    @functools.partial(
        pl.kernel,
        mesh=mesh,
        out_type=jax.ShapeDtypeStruct((B * S * D,), jnp.float32),
        scratch_types=[
            pltpu.VMEM((chunk,), jnp.float32),
            pltpu.VMEM((chunk,), jnp.float32),
        ],
    )
    def sc_add(x_hbm, pe_hbm, out_hbm, xbuf, pebuf):
        cid = lax.axis_index("c")
        sid = lax.axis_index("s")
        wid = sid * 2 + cid
        s_base = wid * rows_per_w

        def chunk_body(c, carry):
            r0 = s_base + c * _R
            pltpu.sync_copy(pe_hbm.at[pl.ds(r0 * D, chunk)], pebuf)

            def b_body(b, carry2):
                off = (b * S + r0) * D
                pltpu.sync_copy(x_hbm.at[pl.ds(off, chunk)], xbuf)

                def add_body(i, carry3):
                    j = i * 16
                    xbuf[pl.ds(j, 16)] = xbuf[pl.ds(j, 16)] + pebuf[pl.ds(j, 16)]
                    return carry3

                lax.fori_loop(0, chunk // 16, add_body, 0)
                pltpu.sync_copy(xbuf, out_hbm.at[pl.ds(off, chunk)])
                return carry2

            return lax.fori_loop(0, B, b_body, carry)

        lax.fori_loop(0, rows_per_w // _R, chunk_body, 0)

    out = sc_add(x1, pe1)
    return out.reshape(B, S, D)
